# baseline (speedup 1.0000x reference)
import jax
import jax.numpy as jnp
from jax import lax
from jax.experimental import pallas as pl
from jax.experimental.pallas import tpu as pltpu
from jax.experimental.pallas import tpu_sc as plsc

VOCAB = 1000000
D = 64
B = 16384
HIST = 50
NW = 32
BAGS_PER_W = B // NW
GROUP_BAGS = 8
GROUP_IDX = GROUP_BAGS * HIST
GROUPS = BAGS_PER_W // GROUP_BAGS
LANES = 16
DV = D // LANES
NBUF = 2
UNROLL = 5


def _lane_total(x, scratch):
    lanes = lax.iota(jnp.int32, LANES)
    for s in (8, 4, 2, 1):
        scratch[pl.ds(0, LANES)] = x
        x = x + plsc.load_gather(scratch, [lanes ^ s])
    return x


def _worker_body(table, idx_hbm, ctx_hbm, out_hbm,
                 idx2_v, flat_v, ctx_v, row0_v, rows_v, out_v, red_v,
                 sem0, sem1):
    nc = 2
    wid = lax.axis_index("s") * nc + lax.axis_index("c")
    lanes = lax.iota(jnp.int32, LANES)

    pltpu.sync_copy(idx_hbm.at[pl.ds(wid * BAGS_PER_W, BAGS_PER_W)], idx2_v)
    pltpu.sync_copy(ctx_hbm.at[pl.ds(wid * BAGS_PER_W, BAGS_PER_W)], ctx_v)
    pltpu.sync_copy(table.at[pl.ds(0, 1)], row0_v)

    def flatten_group(g, buf):
        def row_body(r, carry):
            dst = buf * GROUP_IDX + r * HIST
            for o in (0, LANES, 2 * LANES, HIST - LANES):
                iv = idx2_v[g * GROUP_BAGS + r, pl.ds(o, LANES)]
                plsc.store_scatter(
                    flat_v, [jnp.full((LANES,), dst + o, jnp.int32) + lanes], iv)
            return carry

        lax.fori_loop(0, GROUP_BAGS, row_body, 0)

    def gather(g, buf, sem):
        flatten_group(g, buf)
        return pltpu.async_copy(
            table.at[flat_v.at[pl.ds(buf * GROUP_IDX, GROUP_IDX)]],
            rows_v.at[buf], sem)

    def wait(buf, sem):
        pltpu.make_async_copy(
            table.at[flat_v.at[pl.ds(0, GROUP_IDX)]],
            rows_v.at[buf], sem).wait()

    r0 = [row0_v[0, pl.ds(j * LANES, LANES)] for j in range(DV)]
    zero = jnp.zeros((LANES,), jnp.float32)

    def process_group(g, buf):
        rows = rows_v.at[buf]

        def bag_body(bb, carry):
            b_local = g * GROUP_BAGS + bb
            base = bb * HIST

            def ent_body(e5, accs):
                a = list(accs)
                for u in range(UNROLL):
                    e = base + e5 * UNROLL + u
                    for j in range(DV):
                        a[j] = a[j] + rows[e, pl.ds(j * LANES, LANES)]
                return tuple(a)

            accs = lax.fori_loop(0, HIST // UNROLL, ent_body,
                                 (zero, zero, zero, zero))
            one = jnp.ones((LANES,), jnp.int32)
            nil = jnp.zeros((LANES,), jnp.int32)
            zc = nil
            for t in range(3):
                iv = idx2_v[b_local, pl.ds(t * LANES, LANES)]
                zc = zc + jnp.where(iv == 0, one, nil)
            iv = idx2_v[b_local, pl.ds(HIST - LANES, LANES)]
            zc = zc + jnp.where((iv == 0) & (lanes >= 4 * LANES - HIST),
                                one, nil)
            zf = _lane_total(zc, red_v).astype(jnp.float32)
            cv = plsc.load_gather(
                ctx_v, [jnp.full((LANES,), b_local, jnp.int32)]
            ).astype(jnp.float32)
            inv = 1.0 / cv
            for j in range(DV):
                out_v[b_local, pl.ds(j * LANES, LANES)] = (accs[j] - zf * r0[j]) * inv
            return carry

        lax.fori_loop(0, GROUP_BAGS, bag_body, 0)

    sems = (sem0, sem1)
    for b in range(NBUF - 1):
        gather(b, b, sems[b])

    def step(k, carry):
        for u in range(NBUF):
            g = NBUF * k + u

            @pl.when(g + NBUF - 1 < GROUPS)
            def _():
                gather(g + NBUF - 1, (u + NBUF - 1) % NBUF,
                       sems[(u + NBUF - 1) % NBUF])

            wait(u, sems[u])
            process_group(g, u)
        return carry

    lax.fori_loop(0, GROUPS // NBUF, step, 0)

    pltpu.sync_copy(out_v, out_hbm.at[pl.ds(wid * BAGS_PER_W, BAGS_PER_W)])


@jax.jit
def _cbow_sc(table, idx, ctx):
    mesh = plsc.VectorSubcoreMesh(core_axis_name="c", subcore_axis_name="s")
    f = pl.kernel(
        _worker_body,
        out_type=jax.ShapeDtypeStruct((B, D), jnp.float32),
        mesh=mesh,
        scratch_types=[
            pltpu.VMEM((BAGS_PER_W, HIST), jnp.int32),
            pltpu.VMEM((NBUF * GROUP_IDX,), jnp.int32),
            pltpu.VMEM((BAGS_PER_W,), jnp.int32),
            pltpu.VMEM((1, D), jnp.float32),
            pltpu.VMEM((NBUF, GROUP_IDX, D), jnp.float32),
            pltpu.VMEM((BAGS_PER_W, D), jnp.float32),
            pltpu.VMEM((LANES,), jnp.int32),
            pltpu.SemaphoreType.DMA,
            pltpu.SemaphoreType.DMA,
        ],
        compiler_params=pltpu.CompilerParams(
            needs_layout_passes=False, use_tc_tiling_on_sc=False
        ),
    )
    return f(table, idx, ctx)


def kernel(embedding, bow, context_size):
    return _cbow_sc(embedding, bow.astype(jnp.int32),
                    context_size.astype(jnp.int32))


# speedup vs baseline: 1.0012x; 1.0012x over previous
import jax
import jax.numpy as jnp
from jax import lax
from jax.experimental import pallas as pl
from jax.experimental.pallas import tpu as pltpu
from jax.experimental.pallas import tpu_sc as plsc

VOCAB = 1000000
D = 64
B = 16384
HIST = 50
WIDE = 128
NW = 32
BAGS_PER_W = B // NW
GROUP_BAGS = 8
GROUP_IDX = GROUP_BAGS * HIST
GROUPS = BAGS_PER_W // GROUP_BAGS
HALF_BAGS = BAGS_PER_W // 2
LANES = 16
DV = D // LANES
NBUF = 2
UNROLL = 5


def _lane_total(x, scratch):
    lanes = lax.iota(jnp.int32, LANES)
    for s in (8, 4, 2, 1):
        scratch[pl.ds(0, LANES)] = x
        x = x + plsc.load_gather(scratch, [lanes ^ s])
    return x


def _worker_body(table, idx_hbm, ctx_hbm, out_hbm,
                 idx2_v, flat_v, ctx_v, row0_v, rows_v, out_v, red_v,
                 sem0, sem1):
    nc = 2
    wid = lax.axis_index("s") * nc + lax.axis_index("c")
    lanes = lax.iota(jnp.int32, LANES)

    def stage_half(h):
        pltpu.sync_copy(
            idx_hbm.at[pl.ds(wid * BAGS_PER_W + h * HALF_BAGS, HALF_BAGS)],
            idx2_v)

    stage_half(0)
    pltpu.sync_copy(ctx_hbm.at[pl.ds(wid * BAGS_PER_W, BAGS_PER_W)], ctx_v)
    pltpu.sync_copy(table.at[pl.ds(0, 1)], row0_v)

    def flatten_group(g, buf):
        def row_body(r, carry):
            lr = (g * GROUP_BAGS + r) & (HALF_BAGS - 1)
            dst = buf * GROUP_IDX + r * HIST
            for o in (0, LANES, 2 * LANES, HIST - LANES):
                iv = idx2_v[lr, pl.ds(o, LANES)]
                plsc.store_scatter(
                    flat_v, [jnp.full((LANES,), dst + o, jnp.int32) + lanes], iv)
            return carry

        lax.fori_loop(0, GROUP_BAGS, row_body, 0)

    def gather(g, buf, sem):
        flatten_group(g, buf)
        return pltpu.async_copy(
            table.at[flat_v.at[pl.ds(buf * GROUP_IDX, GROUP_IDX)]],
            rows_v.at[buf], sem)

    def wait(buf, sem):
        pltpu.make_async_copy(
            table.at[flat_v.at[pl.ds(0, GROUP_IDX)]],
            rows_v.at[buf], sem).wait()

    r0 = [row0_v[0, pl.ds(j * LANES, LANES)] for j in range(DV)]
    zero = jnp.zeros((LANES,), jnp.float32)

    def process_group(g, buf):
        rows = rows_v.at[buf]

        def bag_body(bb, carry):
            b_local = g * GROUP_BAGS + bb
            base = bb * HIST

            def ent_body(e5, accs):
                a = list(accs)
                for u in range(UNROLL):
                    e = base + e5 * UNROLL + u
                    for j in range(DV):
                        a[j] = a[j] + rows[e, pl.ds(j * LANES, LANES)]
                return tuple(a)

            accs = lax.fori_loop(0, HIST // UNROLL, ent_body,
                                 (zero, zero, zero, zero))
            one = jnp.ones((LANES,), jnp.int32)
            nil = jnp.zeros((LANES,), jnp.int32)
            zc = nil
            for t in range(3):
                iv = flat_v[pl.ds(buf * GROUP_IDX + base + t * LANES, LANES)]
                zc = zc + jnp.where(iv == 0, one, nil)
            iv = flat_v[pl.ds(buf * GROUP_IDX + base + HIST - LANES, LANES)]
            zc = zc + jnp.where((iv == 0) & (lanes >= 4 * LANES - HIST),
                                one, nil)
            zf = _lane_total(zc, red_v).astype(jnp.float32)
            cv = plsc.load_gather(
                ctx_v, [jnp.full((LANES,), b_local, jnp.int32)]
            ).astype(jnp.float32)
            inv = 1.0 / cv
            for j in range(DV):
                out_v[b_local, pl.ds(j * LANES, LANES)] = (accs[j] - zf * r0[j]) * inv
            return carry

        lax.fori_loop(0, GROUP_BAGS, bag_body, 0)

    sems = (sem0, sem1)
    for b in range(NBUF - 1):
        gather(b, b, sems[b])

    def step(k, carry):
        for u in range(NBUF):
            g = NBUF * k + u

            @pl.when(g + NBUF - 1 == GROUPS // 2)
            def _():
                stage_half(1)

            @pl.when(g + NBUF - 1 < GROUPS)
            def _():
                gather(g + NBUF - 1, (u + NBUF - 1) % NBUF,
                       sems[(u + NBUF - 1) % NBUF])

            wait(u, sems[u])
            process_group(g, u)
        return carry

    lax.fori_loop(0, GROUPS // NBUF, step, 0)

    pltpu.sync_copy(out_v, out_hbm.at[pl.ds(wid * BAGS_PER_W, BAGS_PER_W)])


@jax.jit
def _cbow_sc(table, idx, ctx):
    mesh = plsc.VectorSubcoreMesh(core_axis_name="c", subcore_axis_name="s")
    f = pl.kernel(
        _worker_body,
        out_type=jax.ShapeDtypeStruct((B, D), jnp.float32),
        mesh=mesh,
        scratch_types=[
            pltpu.VMEM((HALF_BAGS, WIDE), jnp.int32),
            pltpu.VMEM((NBUF * GROUP_IDX,), jnp.int32),
            pltpu.VMEM((BAGS_PER_W,), jnp.int32),
            pltpu.VMEM((1, D), jnp.float32),
            pltpu.VMEM((NBUF, GROUP_IDX, D), jnp.float32),
            pltpu.VMEM((BAGS_PER_W, D), jnp.float32),
            pltpu.VMEM((LANES,), jnp.int32),
            pltpu.SemaphoreType.DMA,
            pltpu.SemaphoreType.DMA,
        ],
        compiler_params=pltpu.CompilerParams(
            needs_layout_passes=False, use_tc_tiling_on_sc=False
        ),
    )
    return f(table, idx, ctx)


def kernel(embedding, bow, context_size):
    idx = jnp.pad(bow.astype(jnp.int32), ((0, 0), (0, WIDE - HIST)))
    return _cbow_sc(embedding, idx, context_size.astype(jnp.int32))


# speedup vs baseline: 1.0582x; 1.0570x over previous
import jax
import jax.numpy as jnp
from jax import lax
from jax.experimental import pallas as pl
from jax.experimental.pallas import tpu as pltpu
from jax.experimental.pallas import tpu_sc as plsc

VOCAB = 1000000
D = 64
B = 16384
HIST = 50
WIDE = 128
NW = 32
BAGS_PER_W = B // NW
GROUP_BAGS = 4
GROUP_IDX = GROUP_BAGS * HIST
GROUPS = BAGS_PER_W // GROUP_BAGS
HALF_BAGS = BAGS_PER_W // 2
LANES = 16
DV = D // LANES
NBUF = 2
UNROLL = 5


def _lane_total(x, scratch):
    lanes = lax.iota(jnp.int32, LANES)
    for s in (8, 4, 2, 1):
        scratch[pl.ds(0, LANES)] = x
        x = x + plsc.load_gather(scratch, [lanes ^ s])
    return x


def _worker_body(table, idx_hbm, ctx_hbm, out_hbm,
                 idx2_v, flat_v, ctx_v, row0_v, rows_v, out_v, red_v,
                 sem0, sem1):
    nc = 2
    wid = lax.axis_index("s") * nc + lax.axis_index("c")
    lanes = lax.iota(jnp.int32, LANES)

    def stage_half(h):
        pltpu.sync_copy(
            idx_hbm.at[pl.ds(wid * BAGS_PER_W + h * HALF_BAGS, HALF_BAGS)],
            idx2_v)

    stage_half(0)
    pltpu.sync_copy(ctx_hbm.at[pl.ds(wid * BAGS_PER_W, BAGS_PER_W)], ctx_v)
    pltpu.sync_copy(table.at[pl.ds(0, 1)], row0_v)

    def flatten_group(g, buf):
        def row_body(r, carry):
            lr = (g * GROUP_BAGS + r) & (HALF_BAGS - 1)
            dst = buf * GROUP_IDX + r * HIST
            for o in (0, LANES, 2 * LANES, HIST - LANES):
                iv = idx2_v[lr, pl.ds(o, LANES)]
                plsc.store_scatter(
                    flat_v, [jnp.full((LANES,), dst + o, jnp.int32) + lanes], iv)
            return carry

        lax.fori_loop(0, GROUP_BAGS, row_body, 0)

    def gather(g, buf, sem):
        flatten_group(g, buf)
        return pltpu.async_copy(
            table.at[flat_v.at[pl.ds(buf * GROUP_IDX, GROUP_IDX)]],
            rows_v.at[buf], sem)

    def wait(buf, sem):
        pltpu.make_async_copy(
            table.at[flat_v.at[pl.ds(0, GROUP_IDX)]],
            rows_v.at[buf], sem).wait()

    r0 = [row0_v[0, pl.ds(j * LANES, LANES)] for j in range(DV)]
    zero = jnp.zeros((LANES,), jnp.float32)

    def process_group(g, buf):
        rows = rows_v.at[buf]

        def bag_body(bb, carry):
            b_local = g * GROUP_BAGS + bb
            base = bb * HIST

            def ent_body(e5, accs):
                a = list(accs)
                for u in range(UNROLL):
                    e = base + e5 * UNROLL + u
                    for j in range(DV):
                        a[j] = a[j] + rows[e, pl.ds(j * LANES, LANES)]
                return tuple(a)

            accs = lax.fori_loop(0, HIST // UNROLL, ent_body,
                                 (zero, zero, zero, zero))
            one = jnp.ones((LANES,), jnp.int32)
            nil = jnp.zeros((LANES,), jnp.int32)
            zc = nil
            for t in range(3):
                iv = flat_v[pl.ds(buf * GROUP_IDX + base + t * LANES, LANES)]
                zc = zc + jnp.where(iv == 0, one, nil)
            iv = flat_v[pl.ds(buf * GROUP_IDX + base + HIST - LANES, LANES)]
            zc = zc + jnp.where((iv == 0) & (lanes >= 4 * LANES - HIST),
                                one, nil)
            zf = _lane_total(zc, red_v).astype(jnp.float32)
            cv = plsc.load_gather(
                ctx_v, [jnp.full((LANES,), b_local, jnp.int32)]
            ).astype(jnp.float32)
            inv = 1.0 / cv
            for j in range(DV):
                out_v[b_local, pl.ds(j * LANES, LANES)] = (accs[j] - zf * r0[j]) * inv
            return carry

        lax.fori_loop(0, GROUP_BAGS, bag_body, 0)

    sems = (sem0, sem1)
    for b in range(NBUF - 1):
        gather(b, b, sems[b])

    def step(k, carry):
        for u in range(NBUF):
            g = NBUF * k + u

            @pl.when(g + NBUF - 1 == GROUPS // 2)
            def _():
                stage_half(1)

            @pl.when(g + NBUF - 1 < GROUPS)
            def _():
                gather(g + NBUF - 1, (u + NBUF - 1) % NBUF,
                       sems[(u + NBUF - 1) % NBUF])

            wait(u, sems[u])
            process_group(g, u)
        return carry

    lax.fori_loop(0, GROUPS // NBUF, step, 0)

    pltpu.sync_copy(out_v, out_hbm.at[pl.ds(wid * BAGS_PER_W, BAGS_PER_W)])


@jax.jit
def _cbow_sc(table, idx, ctx):
    mesh = plsc.VectorSubcoreMesh(core_axis_name="c", subcore_axis_name="s")
    f = pl.kernel(
        _worker_body,
        out_type=jax.ShapeDtypeStruct((B, D), jnp.float32),
        mesh=mesh,
        scratch_types=[
            pltpu.VMEM((HALF_BAGS, WIDE), jnp.int32),
            pltpu.VMEM((NBUF * GROUP_IDX,), jnp.int32),
            pltpu.VMEM((BAGS_PER_W,), jnp.int32),
            pltpu.VMEM((1, WIDE), jnp.float32),
            pltpu.VMEM((NBUF, GROUP_IDX, WIDE), jnp.float32),
            pltpu.VMEM((BAGS_PER_W, D), jnp.float32),
            pltpu.VMEM((LANES,), jnp.int32),
            pltpu.SemaphoreType.DMA,
            pltpu.SemaphoreType.DMA,
        ],
        compiler_params=pltpu.CompilerParams(
            needs_layout_passes=False, use_tc_tiling_on_sc=False
        ),
    )
    return f(table, idx, ctx)


def kernel(embedding, bow, context_size):
    emb = jnp.pad(embedding, ((0, 0), (0, WIDE - D)))
    idx = jnp.pad(bow.astype(jnp.int32), ((0, 0), (0, WIDE - HIST)))
    return _cbow_sc(emb, idx, context_size.astype(jnp.int32))
